# baseline (speedup 1.0000x reference)
import functools

import jax
import jax.numpy as jnp
from jax import lax
from jax.experimental import pallas as pl
from jax.experimental.pallas import tpu as pltpu
from jax.experimental.pallas import tpu_sc as plsc

COMPLEXES, RESX, RESY, CH = 2048, 32, 32, 16
RATE2 = 256
N = COMPLEXES * RATE2
NC, NS, L = 2, 16, 16
NW = NC * NS
CPW = COMPLEXES // NW
SPW = N // NW
TEX = RESX * RESY * CH
TEXP = TEX // 2
CHP = CH // 2
GROUPS = RATE2 // L
OW = RATE2 * CH


def _body(tex, u_hbm, v_hbm, out_hbm,
          u_v, v_v, tab0, tab1, o0, o1, ts0, ts1, os0, os1):
    wid = lax.axis_index("s") * NC + lax.axis_index("c")
    s0 = wid * SPW
    c0 = wid * CPW

    pltpu.sync_copy(u_hbm.at[pl.ds(s0, SPW)], u_v)
    pltpu.sync_copy(v_hbm.at[pl.ds(s0, SPW)], v_v)

    lane = lax.broadcasted_iota(jnp.int32, (L,), 0)
    tabs, outs = (tab0, tab1), (o0, o1)
    tsems, osems = (ts0, ts1), (os0, os1)

    pltpu.async_copy(tex.at[c0 + 0], tab0, ts0)
    pltpu.async_copy(tex.at[c0 + 1], tab1, ts1)

    @pl.loop(0, CPW, step=2)
    def _pair(cbase):
        for b in range(2):
            ci = cbase + b
            tab, ov, ts, osem = tabs[b], outs[b], tsems[b], osems[b]
            pltpu.make_async_copy(tex.at[c0 + ci], tab, ts).wait()

            @pl.when(cbase > 0)
            def _wait_out():
                pltpu.make_async_copy(
                    ov, out_hbm.at[:, pl.ds((c0 + ci - 2) * RATE2, RATE2)],
                    osem).wait()

            @plsc.parallel_loop(0, GROUPS, unroll=2)
            def _group(g):
                sb = ci * RATE2 + g * L
                uu = jnp.clip(u_v[pl.ds(sb, L)], 0.0, 1.0)
                vv = jnp.clip(v_v[pl.ds(sb, L)], 0.0, 1.0)
                x = uu * float(RESX - 1)
                y = vv * float(RESY - 1)
                x0 = x.astype(jnp.int32)
                y0 = y.astype(jnp.int32)
                wx = x - x0.astype(jnp.float32)
                wy = y - y0.astype(jnp.float32)
                x1 = jnp.minimum(x0 + 1, RESX - 1)
                y1 = jnp.minimum(y0 + 1, RESY - 1)
                x0r = x0 * (RESY * CHP)
                x1r = x1 * (RESY * CHP)
                y0c = y0 * CHP
                y1c = y1 * CHP
                b00 = x0r + y0c
                b01 = x0r + y1c
                b10 = x1r + y0c
                b11 = x1r + y1c
                wA = (1.0 - wx) * (1.0 - wy)
                wB = (1.0 - wx) * wy
                wC = wx * (1.0 - wy)
                wD = wx * wy
                @plsc.parallel_loop(0, CHP, unroll=2)
                def _chan(chp):
                    p00 = plsc.load_gather(tab, [b00 + chp])
                    p01 = plsc.load_gather(tab, [b01 + chp])
                    p10 = plsc.load_gather(tab, [b10 + chp])
                    p11 = plsc.load_gather(tab, [b11 + chp])
                    himask = jnp.int32(-65536)
                    l00 = plsc.bitcast(p00 << 16, jnp.float32)
                    l01 = plsc.bitcast(p01 << 16, jnp.float32)
                    l10 = plsc.bitcast(p10 << 16, jnp.float32)
                    l11 = plsc.bitcast(p11 << 16, jnp.float32)
                    h00 = plsc.bitcast(p00 & himask, jnp.float32)
                    h01 = plsc.bitcast(p01 & himask, jnp.float32)
                    h10 = plsc.bitcast(p10 & himask, jnp.float32)
                    h11 = plsc.bitcast(p11 & himask, jnp.float32)
                    olo = l00 * wA + l01 * wB + l10 * wC + l11 * wD
                    ohi = h00 * wA + h01 * wB + h10 * wC + h11 * wD
                    ov[2 * chp, pl.ds(g * L, L)] = olo
                    ov[2 * chp + 1, pl.ds(g * L, L)] = ohi

            pltpu.async_copy(
                ov, out_hbm.at[:, pl.ds((c0 + ci) * RATE2, RATE2)], osem)

            @pl.when(ci + 2 < CPW)
            def _prefetch():
                pltpu.async_copy(tex.at[c0 + ci + 2], tab, ts)

    pltpu.make_async_copy(
        o0, out_hbm.at[:, pl.ds((c0 + CPW - 2) * RATE2, RATE2)], os0).wait()
    pltpu.make_async_copy(
        o1, out_hbm.at[:, pl.ds((c0 + CPW - 1) * RATE2, RATE2)], os1).wait()


_mesh = plsc.VectorSubcoreMesh(
    core_axis_name="c", subcore_axis_name="s", num_cores=NC, num_subcores=NS
)

_fetch_sc = functools.partial(
    pl.kernel,
    out_type=jax.ShapeDtypeStruct((CH, N), jnp.float32),
    mesh=_mesh,
    compiler_params=pltpu.CompilerParams(needs_layout_passes=False),
    scratch_types=[
        pltpu.VMEM((SPW,), jnp.float32),
        pltpu.VMEM((SPW,), jnp.float32),
        pltpu.VMEM((TEXP,), jnp.int32),
        pltpu.VMEM((TEXP,), jnp.int32),
        pltpu.VMEM((CH, RATE2), jnp.float32),
        pltpu.VMEM((CH, RATE2), jnp.float32),
        pltpu.SemaphoreType.DMA,
        pltpu.SemaphoreType.DMA,
        pltpu.SemaphoreType.DMA,
        pltpu.SemaphoreType.DMA,
    ],
)(_body)


def kernel(map, u, v):
    mb = map.astype(jnp.bfloat16).reshape(COMPLEXES, TEXP, 2)
    packed = jax.lax.bitcast_convert_type(mb, jnp.int32)
    return _fetch_sc(packed, u, v).T


# speedup vs baseline: 1.8905x; 1.8905x over previous
import functools

import jax
import jax.numpy as jnp
from jax import lax
from jax.experimental import pallas as pl
from jax.experimental.pallas import tpu as pltpu
from jax.experimental.pallas import tpu_sc as plsc

COMPLEXES, RESX, RESY, CH = 2048, 32, 32, 16
RATE2 = 256
N = COMPLEXES * RATE2
NC, NS, L = 2, 16, 16
NW = NC * NS
CPW = COMPLEXES // NW
SPW = N // NW
TEX = RESX * RESY * CH
TEXP = TEX // 2
CHP = CH // 2
GROUPS = RATE2 // L
OW = RATE2 * CH


def _body(tex, u_hbm, v_hbm, out_hbm,
          u_v, v_v, tab0, tab1, o0, o1, ts0, ts1, os0, os1):
    wid = lax.axis_index("s") * NC + lax.axis_index("c")
    s0 = wid * SPW
    c0 = wid * CPW

    pltpu.sync_copy(u_hbm.at[pl.ds(s0, SPW)], u_v)
    pltpu.sync_copy(v_hbm.at[pl.ds(s0, SPW)], v_v)

    lane = lax.broadcasted_iota(jnp.int32, (L,), 0)
    tabs, outs = (tab0, tab1), (o0, o1)
    tsems, osems = (ts0, ts1), (os0, os1)

    pltpu.async_copy(tex.at[c0 + 0], tab0, ts0)
    pltpu.async_copy(tex.at[c0 + 1], tab1, ts1)

    @pl.loop(0, CPW, step=2)
    def _pair(cbase):
        for b in range(2):
            ci = cbase + b
            tab, ov, ts, osem = tabs[b], outs[b], tsems[b], osems[b]
            pltpu.make_async_copy(tex.at[c0 + ci], tab, ts).wait()

            @pl.when(cbase > 0)
            def _wait_out():
                pltpu.make_async_copy(
                    ov, out_hbm.at[:, pl.ds((c0 + ci - 2) * RATE2, RATE2)],
                    osem).wait()

            @plsc.parallel_loop(0, GROUPS, unroll=4)
            def _group(g):
                sb = ci * RATE2 + g * L
                uu = jnp.clip(u_v[pl.ds(sb, L)], 0.0, 1.0)
                vv = jnp.clip(v_v[pl.ds(sb, L)], 0.0, 1.0)
                x = uu * float(RESX - 1)
                y = vv * float(RESY - 1)
                x0 = x.astype(jnp.int32)
                y0 = y.astype(jnp.int32)
                wx = x - x0.astype(jnp.float32)
                wy = y - y0.astype(jnp.float32)
                x1 = jnp.minimum(x0 + 1, RESX - 1)
                y1 = jnp.minimum(y0 + 1, RESY - 1)
                x0r = x0 * (RESY * CH)
                x1r = x1 * (RESY * CH)
                y0c = y0 * CH
                y1c = y1 * CH
                b00 = x0r + y0c
                b01 = x0r + y1c
                b10 = x1r + y0c
                b11 = x1r + y1c
                wA = (1.0 - wx) * (1.0 - wy)
                wB = (1.0 - wx) * wy
                wC = wx * (1.0 - wy)
                wD = wx * wy
                @plsc.parallel_loop(0, CH, unroll=2)
                def _chan(ch):
                    f00 = plsc.load_gather(tab, [b00 + ch])
                    f01 = plsc.load_gather(tab, [b01 + ch])
                    f10 = plsc.load_gather(tab, [b10 + ch])
                    f11 = plsc.load_gather(tab, [b11 + ch])
                    o = f00 * wA + f01 * wB + f10 * wC + f11 * wD
                    ov[ch, pl.ds(g * L, L)] = o

            pltpu.async_copy(
                ov, out_hbm.at[:, pl.ds((c0 + ci) * RATE2, RATE2)], osem)

            @pl.when(ci + 2 < CPW)
            def _prefetch():
                pltpu.async_copy(tex.at[c0 + ci + 2], tab, ts)

    pltpu.make_async_copy(
        o0, out_hbm.at[:, pl.ds((c0 + CPW - 2) * RATE2, RATE2)], os0).wait()
    pltpu.make_async_copy(
        o1, out_hbm.at[:, pl.ds((c0 + CPW - 1) * RATE2, RATE2)], os1).wait()


_mesh = plsc.VectorSubcoreMesh(
    core_axis_name="c", subcore_axis_name="s", num_cores=NC, num_subcores=NS
)

_fetch_sc = functools.partial(
    pl.kernel,
    out_type=jax.ShapeDtypeStruct((CH, N), jnp.float32),
    mesh=_mesh,
    compiler_params=pltpu.CompilerParams(needs_layout_passes=False),
    scratch_types=[
        pltpu.VMEM((SPW,), jnp.float32),
        pltpu.VMEM((SPW,), jnp.float32),
        pltpu.VMEM((TEX,), jnp.float32),
        pltpu.VMEM((TEX,), jnp.float32),
        pltpu.VMEM((CH, RATE2), jnp.float32),
        pltpu.VMEM((CH, RATE2), jnp.float32),
        pltpu.SemaphoreType.DMA,
        pltpu.SemaphoreType.DMA,
        pltpu.SemaphoreType.DMA,
        pltpu.SemaphoreType.DMA,
    ],
)(_body)


def kernel(map, u, v):
    return _fetch_sc(map.reshape(COMPLEXES, TEX), u, v).T


# speedup vs baseline: 1.9195x; 1.0154x over previous
import functools

import jax
import jax.numpy as jnp
from jax import lax
from jax.experimental import pallas as pl
from jax.experimental.pallas import tpu as pltpu
from jax.experimental.pallas import tpu_sc as plsc

COMPLEXES, RESX, RESY, CH = 2048, 32, 32, 16
RATE2 = 256
N = COMPLEXES * RATE2
NC, NS, L = 2, 16, 16
NW = NC * NS
CPW = COMPLEXES // NW
SPW = N // NW
TEX = RESX * RESY * CH
TEXP = TEX // 2
CHP = CH // 2
GROUPS = RATE2 // L
OW = RATE2 * CH


def _body(tex, u_hbm, v_hbm, out_hbm,
          u_v, v_v, tab0, tab1, o0, o1, ts0, ts1, os0, os1):
    wid = lax.axis_index("s") * NC + lax.axis_index("c")
    s0 = wid * SPW
    c0 = wid * CPW

    pltpu.sync_copy(u_hbm.at[pl.ds(s0, SPW)], u_v)
    pltpu.sync_copy(v_hbm.at[pl.ds(s0, SPW)], v_v)

    lane = lax.broadcasted_iota(jnp.int32, (L,), 0)
    tabs, outs = (tab0, tab1), (o0, o1)
    tsems, osems = (ts0, ts1), (os0, os1)

    pltpu.async_copy(tex.at[c0 + 0], tab0, ts0)
    pltpu.async_copy(tex.at[c0 + 1], tab1, ts1)

    @pl.loop(0, CPW, step=2)
    def _pair(cbase):
        for b in range(2):
            ci = cbase + b
            tab, ov, ts, osem = tabs[b], outs[b], tsems[b], osems[b]
            pltpu.make_async_copy(tex.at[c0 + ci], tab, ts).wait()

            @pl.when(cbase > 0)
            def _wait_out():
                pltpu.make_async_copy(
                    ov, out_hbm.at[:, pl.ds((c0 + ci - 2) * RATE2, RATE2)],
                    osem).wait()

            @plsc.parallel_loop(0, GROUPS, unroll=8)
            def _group(g):
                sb = ci * RATE2 + g * L
                uu = jnp.clip(u_v[pl.ds(sb, L)], 0.0, 1.0)
                vv = jnp.clip(v_v[pl.ds(sb, L)], 0.0, 1.0)
                x = uu * float(RESX - 1)
                y = vv * float(RESY - 1)
                x0 = x.astype(jnp.int32)
                y0 = y.astype(jnp.int32)
                wx = x - x0.astype(jnp.float32)
                wy = y - y0.astype(jnp.float32)
                x1 = jnp.minimum(x0 + 1, RESX - 1)
                y1 = jnp.minimum(y0 + 1, RESY - 1)
                x0r = x0 * (RESY * CH)
                x1r = x1 * (RESY * CH)
                y0c = y0 * CH
                y1c = y1 * CH
                b00 = x0r + y0c
                b01 = x0r + y1c
                b10 = x1r + y0c
                b11 = x1r + y1c
                wA = (1.0 - wx) * (1.0 - wy)
                wB = (1.0 - wx) * wy
                wC = wx * (1.0 - wy)
                wD = wx * wy
                @plsc.parallel_loop(0, CH, unroll=2)
                def _chan(ch):
                    f00 = plsc.load_gather(tab, [b00 + ch])
                    f01 = plsc.load_gather(tab, [b01 + ch])
                    f10 = plsc.load_gather(tab, [b10 + ch])
                    f11 = plsc.load_gather(tab, [b11 + ch])
                    o = f00 * wA + f01 * wB + f10 * wC + f11 * wD
                    ov[ch, pl.ds(g * L, L)] = o

            pltpu.async_copy(
                ov, out_hbm.at[:, pl.ds((c0 + ci) * RATE2, RATE2)], osem)

            @pl.when(ci + 2 < CPW)
            def _prefetch():
                pltpu.async_copy(tex.at[c0 + ci + 2], tab, ts)

    pltpu.make_async_copy(
        o0, out_hbm.at[:, pl.ds((c0 + CPW - 2) * RATE2, RATE2)], os0).wait()
    pltpu.make_async_copy(
        o1, out_hbm.at[:, pl.ds((c0 + CPW - 1) * RATE2, RATE2)], os1).wait()


_mesh = plsc.VectorSubcoreMesh(
    core_axis_name="c", subcore_axis_name="s", num_cores=NC, num_subcores=NS
)

_fetch_sc = functools.partial(
    pl.kernel,
    out_type=jax.ShapeDtypeStruct((CH, N), jnp.float32),
    mesh=_mesh,
    compiler_params=pltpu.CompilerParams(needs_layout_passes=False),
    scratch_types=[
        pltpu.VMEM((SPW,), jnp.float32),
        pltpu.VMEM((SPW,), jnp.float32),
        pltpu.VMEM((TEX,), jnp.float32),
        pltpu.VMEM((TEX,), jnp.float32),
        pltpu.VMEM((CH, RATE2), jnp.float32),
        pltpu.VMEM((CH, RATE2), jnp.float32),
        pltpu.SemaphoreType.DMA,
        pltpu.SemaphoreType.DMA,
        pltpu.SemaphoreType.DMA,
        pltpu.SemaphoreType.DMA,
    ],
)(_body)


def kernel(map, u, v):
    return _fetch_sc(map.reshape(COMPLEXES, TEX), u, v).T
